# baseline (speedup 1.0000x reference)
import functools

import jax
import jax.numpy as jnp
from jax import lax
from jax.experimental import pallas as pl
from jax.experimental.pallas import tpu as pltpu
from jax.experimental.pallas import tpu_sc as plsc

N = 10000
D = 256
H = 8
HD = 32
TW = 384

NW = 32
RPT = 320
NPAD = NW * RPT
CH = 48
CPB = 8
BL = CH * CPB
META = BL + 16
EPAD = 512


def _prep_body(x_ref, ws_ref, bsb_ref, wn_ref, bnb_ref, ams_ref, amn_ref,
               tab_ref, atts_ref):
    x = x_ref[...]
    fs = jnp.maximum(
        jnp.dot(x, ws_ref[...], preferred_element_type=jnp.float32)
        + bsb_ref[...], 0.0)
    fn = jnp.maximum(
        jnp.dot(x, wn_ref[...], preferred_element_type=jnp.float32)
        + bnb_ref[...], 0.0)
    as_raw = jnp.dot(fs, ams_ref[...], preferred_element_type=jnp.float32)
    an_raw = jnp.dot(fs, amn_ref[...], preferred_element_type=jnp.float32)
    attn = jnp.where(an_raw >= 0.0, an_raw, 0.2 * an_raw)
    tab_ref[...] = jnp.concatenate([fn, attn], axis=1)
    atts_ref[...] = jnp.where(as_raw >= 0.0, as_raw, 0.2 * as_raw)


def _tc_prep(x, ws_all, bs_all, wn_all, bn_all, ams, amn):
    bm = 1000
    grid = N // bm
    return pl.pallas_call(
        _prep_body,
        grid=(grid,),
        in_specs=[
            pl.BlockSpec((bm, D), lambda i: (i, 0)),
            pl.BlockSpec((D, D), lambda i: (0, 0)),
            pl.BlockSpec((1, D), lambda i: (0, 0)),
            pl.BlockSpec((D, D), lambda i: (0, 0)),
            pl.BlockSpec((1, D), lambda i: (0, 0)),
            pl.BlockSpec((D, 128), lambda i: (0, 0)),
            pl.BlockSpec((D, 128), lambda i: (0, 0)),
        ],
        out_specs=[
            pl.BlockSpec((bm, TW), lambda i: (i, 0)),
            pl.BlockSpec((bm, 128), lambda i: (i, 0)),
        ],
        out_shape=[
            jax.ShapeDtypeStruct((N, TW), jnp.float32),
            jax.ShapeDtypeStruct((N, 128), jnp.float32),
        ],
    )(x, ws_all, bs_all, wn_all, bn_all, ams, amn)


def _bn_body(x_ref, g_ref, b_ref, o_ref):
    x = x_ref[...]
    mu = jnp.sum(x, axis=0, keepdims=True) * (1.0 / N)
    d = x - mu
    var = (jnp.sum(d * d, axis=0, keepdims=True)
           - float(NPAD - N) * mu * mu) * (1.0 / N)
    y = d / jnp.sqrt(var + 1e-9) * g_ref[...] + b_ref[...]
    o_ref[...] = y[:N, :]


def _tc_bn(x, gamma, beta):
    return pl.pallas_call(
        _bn_body,
        out_shape=jax.ShapeDtypeStruct((N, D), jnp.float32),
    )(x, gamma.reshape(1, D), beta.reshape(1, D))


def _sc_agg_body(t_hbm, as_hbm, col_hbm, rv_hbm, st_hbm, out_hbm,
                 as_v, colb_v, rvb_v, taba_v, tabb_v, acc_v, st_v,
                 sema, semb):
    wid = lax.axis_index("s") * 2 + lax.axis_index("c")
    base_row = wid * RPT

    pltpu.sync_copy(st_hbm, st_v)
    sts = st_v[pl.ds(wid, 16)]
    e0 = sts[0]
    e1 = sts[1]

    pltpu.sync_copy(as_hbm.at[pl.ds(base_row * H, (RPT + 1) * H)], as_v)

    zvec = jnp.zeros((16,), jnp.float32)

    def zbody(r, carry):
        for sl in range(16):
            acc_v[pl.ds(r * D + sl * 16, 16)] = zvec
        return carry

    lax.fori_loop(0, RPT, zbody, 0)

    e0a = (e0 // 8) * 8
    nb = (e1 - e0a + (BL - 1)) // BL

    def edge_step(j, ec, cur_tab, ebase, moff):
        rp = ec[0]
        regs = ec[1:]
        e = ebase + j
        m = rvb_v[pl.ds(2 * (moff + j), 16)]
        rraw = m[0]
        vraw = lax.bitcast_convert_type(m[1], jnp.float32)
        valid = jnp.logical_and(e >= e0, e < e1)
        r = jnp.where(valid, rraw, rp)
        vv = jnp.where(valid, vraw, 0.0)
        changed = jnp.not_equal(r, rp)

        @pl.when(changed)
        def _():
            rl = rp - base_row
            for sidx in range(16):
                acc_v[pl.ds(rl * D + sidx * 16, 16)] = regs[sidx]

        asv = as_v[pl.ds((r - base_row) * H, 16)]
        anv = cur_tab[j, pl.ds(D, 16)]
        att = (asv + anv) * vv
        new_regs = []
        for h in range(H):
            ah = att[h]
            for k2 in range(2):
                sidx = h * 2 + k2
                sl = h * HD + k2 * 16
                base = jnp.where(changed, zvec, regs[sidx])
                new_regs.append(base + ah * cur_tab[j, pl.ds(sl, 16)])
        return (r,) + tuple(new_regs)

    def bbody(b, carry):
        bbase = e0a + b * BL
        pltpu.sync_copy(col_hbm.at[pl.ds(bbase, BL)], colb_v)
        pltpu.sync_copy(rv_hbm.at[pl.ds(2 * bbase, 2 * META)], rvb_v)
        descs = [pltpu.async_copy(
            t_hbm.at[colb_v.at[pl.ds(0, CH)]], taba_v, sema)]
        for k in range(CPB):
            cur_tab = taba_v if k % 2 == 0 else tabb_v
            if k + 1 < CPB:
                nxt_tab = tabb_v if k % 2 == 0 else taba_v
                nxt_sem = semb if k % 2 == 0 else sema
                descs.append(pltpu.async_copy(
                    t_hbm.at[colb_v.at[pl.ds((k + 1) * CH, CH)]],
                    nxt_tab, nxt_sem))
            descs[k].wait()
            ebase = bbase + k * CH
            moff = k * CH

            def ebody2(i, ec, cur_tab=cur_tab, ebase=ebase, moff=moff):
                ec = edge_step(2 * i, ec, cur_tab, ebase, moff)
                return edge_step(2 * i + 1, ec, cur_tab, ebase, moff)

            carry = lax.fori_loop(0, CH // 2, ebody2, carry)
        return carry

    init = (base_row,) + (zvec,) * 16
    fin = lax.fori_loop(0, nb, bbody, init)
    rl = fin[0] - base_row
    for sidx in range(16):
        acc_v[pl.ds(rl * D + sidx * 16, 16)] = fin[1 + sidx]

    pltpu.sync_copy(acc_v, out_hbm.at[pl.ds(base_row * D, RPT * D)])


def _sc_agg(table, as_flat, col_p, rv_p, st):
    mesh = plsc.VectorSubcoreMesh(core_axis_name="c", subcore_axis_name="s")
    f = functools.partial(
        pl.kernel,
        out_type=jax.ShapeDtypeStruct((NPAD * D,), jnp.float32),
        mesh=mesh,
        scratch_types=[
            pltpu.VMEM(((RPT + 1) * H,), jnp.float32),
            pltpu.VMEM((BL,), jnp.int32),
            pltpu.VMEM((2 * META,), jnp.int32),
            pltpu.VMEM((CH, TW), jnp.float32),
            pltpu.VMEM((CH, TW), jnp.float32),
            pltpu.VMEM((RPT * D,), jnp.float32),
            pltpu.VMEM((48,), jnp.int32),
            pltpu.SemaphoreType.DMA,
            pltpu.SemaphoreType.DMA,
        ],
    )(_sc_agg_body)
    return f(table, as_flat, col_p, rv_p, st)


def _att_mat(a):
    m = jnp.zeros((D, 128), jnp.float32)
    for h in range(H):
        m = m.at[h * HD:(h + 1) * HD, h].set(a[h, :, 0])
    return m


def kernel(f_in, edge_row, edge_col, edge_val, Ws, bs, Wn, bn, a_s, a_n,
           gamma, beta):
    E = edge_row.shape[0]
    col_p = jnp.concatenate([edge_col, jnp.zeros((EPAD,), jnp.int32)])
    row_p = jnp.concatenate([edge_row, jnp.zeros((EPAD,), jnp.int32)])
    val_p = jnp.concatenate([edge_val, jnp.zeros((EPAD,), jnp.float32)])
    rv_p = jnp.stack(
        [row_p, lax.bitcast_convert_type(val_p, jnp.int32)], axis=1
    ).reshape(-1)
    bounds = jnp.arange(NW + 1, dtype=jnp.int32) * RPT
    st = jnp.searchsorted(edge_row, bounds, side="left").astype(jnp.int32)
    st = jnp.concatenate([st, jnp.zeros((48 - NW - 1,), jnp.int32)])

    x = f_in
    L = Ws.shape[0]
    for i in range(L):
        ws_all = Ws[i].transpose(1, 0, 2).reshape(D, D)
        wn_all = Wn[i].transpose(1, 0, 2).reshape(D, D)
        bs_all = bs[i].reshape(1, D)
        bn_all = bn[i].reshape(1, D)
        ams = _att_mat(a_s[i])
        amn = _att_mat(a_n[i])
        table, atts = _tc_prep(x, ws_all, bs_all, wn_all, bn_all, ams, amn)
        as_flat = jnp.concatenate(
            [atts[:, :H], jnp.zeros((NPAD + 1 - N, H), jnp.float32)]
        ).reshape(-1)
        agg = _sc_agg(table, as_flat, col_p, rv_p, st)
        x = _tc_bn(agg.reshape(NPAD, D), gamma[i], beta[i])
    return x


# speedup vs baseline: 1.1504x; 1.1504x over previous
import functools

import jax
import jax.numpy as jnp
from jax import lax
from jax.experimental import pallas as pl
from jax.experimental.pallas import tpu as pltpu
from jax.experimental.pallas import tpu_sc as plsc

N = 10000
D = 256
H = 8
HD = 32
TW = 384

NW = 32
RPT = 320
NPAD = NW * RPT
CH = 48
CPB = 8
BL = CH * CPB
META = BL + 16
EPAD = 512


def _prep_body(x_ref, ws_ref, bsb_ref, wn_ref, bnb_ref, ams_ref, amn_ref,
               tab_ref, atts_ref):
    x = x_ref[...]
    fs = jnp.maximum(
        jnp.dot(x, ws_ref[...], preferred_element_type=jnp.float32)
        + bsb_ref[...], 0.0)
    fn = jnp.maximum(
        jnp.dot(x, wn_ref[...], preferred_element_type=jnp.float32)
        + bnb_ref[...], 0.0)
    as_raw = jnp.dot(fs, ams_ref[...], preferred_element_type=jnp.float32)
    an_raw = jnp.dot(fs, amn_ref[...], preferred_element_type=jnp.float32)
    attn = jnp.where(an_raw >= 0.0, an_raw, 0.2 * an_raw)
    tab_ref[...] = jnp.concatenate([fn, attn], axis=1)
    atts_ref[...] = jnp.where(as_raw >= 0.0, as_raw, 0.2 * as_raw)


def _tc_prep(x, ws_all, bs_all, wn_all, bn_all, ams, amn):
    bm = 1000
    grid = N // bm
    return pl.pallas_call(
        _prep_body,
        grid=(grid,),
        in_specs=[
            pl.BlockSpec((bm, D), lambda i: (i, 0)),
            pl.BlockSpec((D, D), lambda i: (0, 0)),
            pl.BlockSpec((1, D), lambda i: (0, 0)),
            pl.BlockSpec((D, D), lambda i: (0, 0)),
            pl.BlockSpec((1, D), lambda i: (0, 0)),
            pl.BlockSpec((D, 128), lambda i: (0, 0)),
            pl.BlockSpec((D, 128), lambda i: (0, 0)),
        ],
        out_specs=[
            pl.BlockSpec((bm, TW), lambda i: (i, 0)),
            pl.BlockSpec((bm, 128), lambda i: (i, 0)),
        ],
        out_shape=[
            jax.ShapeDtypeStruct((N, TW), jnp.float32),
            jax.ShapeDtypeStruct((N, 128), jnp.float32),
        ],
    )(x, ws_all, bs_all, wn_all, bn_all, ams, amn)


def _bn_body(x_ref, g_ref, b_ref, o_ref):
    x = x_ref[...]
    mu = jnp.sum(x, axis=0, keepdims=True) * (1.0 / N)
    d = x - mu
    var = (jnp.sum(d * d, axis=0, keepdims=True)
           - float(NPAD - N) * mu * mu) * (1.0 / N)
    y = d / jnp.sqrt(var + 1e-9) * g_ref[...] + b_ref[...]
    o_ref[...] = y[:N, :]


def _tc_bn(x, gamma, beta):
    return pl.pallas_call(
        _bn_body,
        out_shape=jax.ShapeDtypeStruct((N, D), jnp.float32),
    )(x, gamma.reshape(1, D), beta.reshape(1, D))


def _sc_agg_body(t_hbm, as_hbm, col_hbm, row_hbm, val_hbm, st_hbm, out_hbm,
                 as_v, colb_v, rowb_v, valb_v, taba_v, tabb_v, acc_v,
                 st_v, sema, semb):
    wid = lax.axis_index("s") * 2 + lax.axis_index("c")
    base_row = wid * RPT

    pltpu.sync_copy(st_hbm, st_v)
    sts = st_v[pl.ds(wid, 16)]
    e0 = sts[0]
    e1 = sts[1]

    pltpu.sync_copy(as_hbm.at[pl.ds(base_row * H, (RPT + 1) * H)], as_v)

    zvec = jnp.zeros((16,), jnp.float32)

    def zbody(r, carry):
        for sl in range(16):
            acc_v[pl.ds(r * D + sl * 16, 16)] = zvec
        return carry

    lax.fori_loop(0, RPT, zbody, 0)

    e0a = (e0 // 8) * 8
    nb = (e1 - e0a + (BL - 1)) // BL

    def edge_step(j, rp, rraw, regs, cur_tab, ebase, moff):
        e = ebase + j
        valid = jnp.logical_and(e >= e0, e < e1)
        r = jnp.where(valid, rraw, rp)
        changed = jnp.not_equal(r, rp)

        @pl.when(changed)
        def _():
            rl = rp - base_row
            for sidx in range(16):
                acc_v[pl.ds(rl * D + sidx * 16, 16)] = regs[sidx]

        valv = valb_v[pl.ds(moff + j, 16)]
        asv = as_v[pl.ds((r - base_row) * H, 16)]
        anv = cur_tab[j, pl.ds(D, 16)]
        att = (asv + anv) * valv[0]
        att = jnp.where(valid, att, zvec)
        new_regs = []
        for h in range(H):
            ah = att[h]
            for k2 in range(2):
                sidx = h * 2 + k2
                sl = h * HD + k2 * 16
                base = jnp.where(changed, zvec, regs[sidx])
                new_regs.append(base + ah * cur_tab[j, pl.ds(sl, 16)])
        return (r,) + tuple(new_regs)

    def bbody(b, carry):
        bbase = e0a + b * BL
        pltpu.sync_copy(col_hbm.at[pl.ds(bbase, BL)], colb_v)
        pltpu.sync_copy(row_hbm.at[pl.ds(bbase, META)], rowb_v)
        pltpu.sync_copy(val_hbm.at[pl.ds(bbase, META)], valb_v)
        descs = [pltpu.async_copy(
            t_hbm.at[colb_v.at[pl.ds(0, CH)]], taba_v, sema)]
        for k in range(CPB):
            cur_tab = taba_v if k % 2 == 0 else tabb_v
            if k + 1 < CPB:
                nxt_tab = tabb_v if k % 2 == 0 else taba_v
                nxt_sem = semb if k % 2 == 0 else sema
                descs.append(pltpu.async_copy(
                    t_hbm.at[colb_v.at[pl.ds((k + 1) * CH, CH)]],
                    nxt_tab, nxt_sem))
            descs[k].wait()
            ebase = bbase + k * CH
            moff = k * CH

            def ebody2(i, ec, cur_tab=cur_tab, ebase=ebase, moff=moff):
                rp, rna, rnb = ec[0], ec[1], ec[2]
                regs = ec[3:]
                rna2 = rowb_v[pl.ds(moff + 2 * i + 2, 16)][0]
                rnb2 = rowb_v[pl.ds(moff + 2 * i + 3, 16)][0]
                st1 = edge_step(2 * i, rp, rna, regs, cur_tab, ebase, moff)
                st2 = edge_step(2 * i + 1, st1[0], rnb, st1[1:], cur_tab,
                                ebase, moff)
                return (st2[0], rna2, rnb2) + tuple(st2[1:])

            carry = lax.fori_loop(0, CH // 2, ebody2, carry)
        return carry

    pltpu.sync_copy(row_hbm.at[pl.ds(e0a, 16)], rowb_v.at[pl.ds(0, 16)])
    r01 = rowb_v[pl.ds(0, 16)]
    init = (base_row, r01[0], r01[1]) + (zvec,) * 16
    fin = lax.fori_loop(0, nb, bbody, init)
    rl = fin[0] - base_row
    for sidx in range(16):
        acc_v[pl.ds(rl * D + sidx * 16, 16)] = fin[3 + sidx]

    pltpu.sync_copy(acc_v, out_hbm.at[pl.ds(base_row * D, RPT * D)])


def _sc_agg(table, as_flat, col_p, row_p, val_p, st):
    mesh = plsc.VectorSubcoreMesh(core_axis_name="c", subcore_axis_name="s")
    f = functools.partial(
        pl.kernel,
        out_type=jax.ShapeDtypeStruct((NPAD * D,), jnp.float32),
        mesh=mesh,
        scratch_types=[
            pltpu.VMEM(((RPT + 1) * H,), jnp.float32),
            pltpu.VMEM((BL,), jnp.int32),
            pltpu.VMEM((META,), jnp.int32),
            pltpu.VMEM((META,), jnp.float32),
            pltpu.VMEM((CH, TW), jnp.float32),
            pltpu.VMEM((CH, TW), jnp.float32),
            pltpu.VMEM((RPT * D,), jnp.float32),
            pltpu.VMEM((48,), jnp.int32),
            pltpu.SemaphoreType.DMA,
            pltpu.SemaphoreType.DMA,
        ],
    )(_sc_agg_body)
    return f(table, as_flat, col_p, row_p, val_p, st)


def _att_mat(a, w):
    m = jnp.zeros((D, w), jnp.float32)
    for h in range(H):
        m = m.at[h * HD:(h + 1) * HD, h].set(a[h, :, 0])
    return m


def kernel(f_in, edge_row, edge_col, edge_val, Ws, bs, Wn, bn, a_s, a_n,
           gamma, beta):
    E = edge_row.shape[0]
    col_p = jnp.concatenate([edge_col, jnp.zeros((EPAD,), jnp.int32)])
    row_p = jnp.concatenate([edge_row, jnp.zeros((EPAD,), jnp.int32)])
    val_p = jnp.concatenate([edge_val, jnp.zeros((EPAD,), jnp.float32)])
    bounds = jnp.arange(NW + 1, dtype=jnp.int32) * RPT
    st = jnp.searchsorted(edge_row, bounds, side="left").astype(jnp.int32)
    st = jnp.concatenate([st, jnp.zeros((48 - NW - 1,), jnp.int32)])

    x = f_in
    L = Ws.shape[0]
    for i in range(L):
        ws_all = Ws[i].transpose(1, 0, 2).reshape(D, D)
        wn_all = Wn[i].transpose(1, 0, 2).reshape(D, D)
        bs_all = bs[i].reshape(1, D)
        bn_all = bn[i].reshape(1, D)
        ams = _att_mat(a_s[i], 128)
        amn = _att_mat(a_n[i], 128)
        table, atts = _tc_prep(x, ws_all, bs_all, wn_all, bn_all, ams, amn)
        as_flat = jnp.concatenate(
            [atts[:, :H], jnp.zeros((NPAD + 1 - N, H), jnp.float32)]
        ).reshape(-1)
        agg = _sc_agg(table, as_flat, col_p, row_p, val_p, st)
        x = _tc_bn(agg.reshape(NPAD, D), gamma[i], beta[i])
    return x
